# baseline (speedup 1.0000x reference)
import functools

import jax
import jax.numpy as jnp
from jax.experimental import pallas as pl
from jax.experimental.pallas import tpu as pltpu

Q = 1024
D = 256
K = 100000
TOPK = 32
BUCKET = 128
KT = 2048
NB = 784
KPAD = NB * BUCKET
NKT = KPAD // KT
BPT = KT // BUCKET


def _proj_body(query_ref, wq_ref, bq_ref, q_ref, qn_ref):
    qp = jax.lax.dot_general(
        query_ref[...], wq_ref[...], (((1,), (1,)), ((), ())),
        preferred_element_type=jnp.float32) + bq_ref[...]
    q_ref[...] = qp
    qn_ref[...] = jnp.sqrt(jnp.sum(qp * qp, axis=1, keepdims=True))


def _scores_body(q_ref, qn_ref, keys_ref, imp_ref, s_ref, m_ref):
    k_idx = pl.program_id(0)
    keys = keys_ref[...]
    num = jax.lax.dot_general(
        q_ref[...], keys, (((1,), (1,)), ((), ())),
        preferred_element_type=jnp.float32)
    kn2 = jax.lax.dot_general(
        jnp.ones((1, D), jnp.float32), keys * keys, (((1,), (1,)), ((), ())),
        preferred_element_type=jnp.float32,
        precision=jax.lax.Precision.HIGHEST)
    kn = jnp.sqrt(kn2)
    denom = jnp.maximum(qn_ref[...] * kn, 1e-8)
    s = (num / denom) * imp_ref[...]
    jglob = k_idx * KT + jax.lax.broadcasted_iota(jnp.int32, (1, KT), 1)
    s = jnp.where(jglob < K, s, -3e38)
    for b in range(BPT):
        blk = s[:, b * BUCKET:(b + 1) * BUCKET]
        s_ref[:, b, :] = blk
        m_ref[0, :, b] = jnp.max(blk, axis=1)


def _scores_call(q, qn, keys_pad, imp_pad):
    return pl.pallas_call(
        _scores_body,
        grid=(NKT,),
        in_specs=[
            pl.BlockSpec((Q, D), lambda k: (0, 0)),
            pl.BlockSpec((Q, 1), lambda k: (0, 0)),
            pl.BlockSpec((KT, D), lambda k: (k, 0)),
            pl.BlockSpec((1, KT), lambda k: (0, k)),
        ],
        out_specs=[
            pl.BlockSpec((Q, BPT, BUCKET), lambda k: (0, k, 0)),
            pl.BlockSpec((1, Q, BPT), lambda k: (k, 0, 0)),
        ],
        out_shape=[
            jax.ShapeDtypeStruct((Q, NB, BUCKET), jnp.float32),
            jax.ShapeDtypeStruct((NKT, Q, BPT), jnp.float32),
        ],
    )(q, qn, keys_pad, imp_pad)


def kernel(query, mem_keys, mem_values, importance, Wq, bq, top_k):
    query = query.astype(jnp.float32)
    mem_keys = mem_keys.astype(jnp.float32)
    bq2 = bq.reshape(1, D).astype(jnp.float32)

    q, qn = pl.pallas_call(
        _proj_body,
        in_specs=[
            pl.BlockSpec((Q, D), lambda: (0, 0)),
            pl.BlockSpec((D, D), lambda: (0, 0)),
            pl.BlockSpec((1, D), lambda: (0, 0)),
        ],
        out_specs=[
            pl.BlockSpec((Q, D), lambda: (0, 0)),
            pl.BlockSpec((Q, 1), lambda: (0, 0)),
        ],
        out_shape=[
            jax.ShapeDtypeStruct((Q, D), jnp.float32),
            jax.ShapeDtypeStruct((Q, 1), jnp.float32),
        ],
    )(query, Wq, bq2)

    keys_pad = jnp.pad(mem_keys, ((0, KPAD - K), (0, 0)))
    imp_pad = jnp.pad(importance.astype(jnp.float32), (0, KPAD - K)).reshape(1, KPAD)

    scores3, m3 = _scores_call(q, qn, keys_pad, imp_pad)
    scores = scores3.reshape(Q, KPAD)

    top_vals, top_idx = jax.lax.top_k(scores, TOPK)
    weights = jax.nn.softmax(top_vals, axis=-1)
    vals = jnp.take(mem_values, top_idx, axis=0)
    retrieved = jnp.sum(vals * weights[..., None], axis=1)
    return retrieved


# speedup vs baseline: 21.3696x; 21.3696x over previous
import functools

import jax
import jax.numpy as jnp
from jax import lax
from jax.experimental import pallas as pl
from jax.experimental.pallas import tpu as pltpu
from jax.experimental.pallas import tpu_sc as plsc

Q = 1024
D = 256
K = 100000
TOPK = 32
BUCKET = 128
KT = 2048
NB = 784
KPAD = NB * BUCKET
NKT = KPAD // KT
BPT = KT // BUCKET


def _proj_body(query_ref, wq_ref, bq_ref, q_ref, qn_ref):
    qp = jax.lax.dot_general(
        query_ref[...], wq_ref[...], (((1,), (1,)), ((), ())),
        preferred_element_type=jnp.float32) + bq_ref[...]
    q_ref[...] = qp
    qn = jnp.sqrt(jnp.sum(qp * qp, axis=1, keepdims=True))
    qn_ref[...] = 1.0 / jnp.maximum(qn, 1e-8)


def _scores_body(q_ref, rqn_ref, keys_ref, imp_ref, s_ref, m_ref):
    k_idx = pl.program_id(0)
    keys = keys_ref[...]
    num = jax.lax.dot_general(
        q_ref[...], keys, (((1,), (1,)), ((), ())),
        preferred_element_type=jnp.float32)
    kn2 = jax.lax.dot_general(
        jnp.ones((1, D), jnp.float32), keys * keys, (((1,), (1,)), ((), ())),
        preferred_element_type=jnp.float32,
        precision=jax.lax.Precision.HIGHEST)
    kn = jnp.sqrt(kn2)
    c = imp_ref[...] / jnp.maximum(kn, 1e-8)
    s = num * rqn_ref[...] * c
    jglob = k_idx * KT + jax.lax.broadcasted_iota(jnp.int32, (1, KT), 1)
    s = jnp.where(jglob < K, s, 0.0)
    cols = []
    for b in range(BPT):
        blk = s[:, b * BUCKET:(b + 1) * BUCKET]
        s_ref[pl.ds(b * Q, Q), :] = blk
        cols.append(jnp.max(blk, axis=1, keepdims=True))
    m_ref[0] = jnp.concatenate(cols, axis=1)


def _scores_call(q, qn, keys_pad, imp_pad):
    return pl.pallas_call(
        _scores_body,
        grid=(NKT,),
        in_specs=[
            pl.BlockSpec((Q, D), lambda k: (0, 0)),
            pl.BlockSpec((Q, 1), lambda k: (0, 0)),
            pl.BlockSpec((KT, D), lambda k: (k, 0)),
            pl.BlockSpec((1, KT), lambda k: (0, k)),
        ],
        out_specs=[
            pl.BlockSpec((BPT * Q, BUCKET), lambda k: (k, 0)),
            pl.BlockSpec((1, Q, BPT), lambda k: (k, 0, 0)),
        ],
        out_shape=[
            jax.ShapeDtypeStruct((NB * Q, BUCKET), jnp.float32),
            jax.ShapeDtypeStruct((NKT, Q, BPT), jnp.float32),
        ],
    )(q, qn, keys_pad, imp_pad)


QB = 256
NEG = -3e38


def _buckets_body(m_ref, bids_ref, tau_ref):
    m = m_ref[...]
    iota = lax.broadcasted_iota(jnp.int32, (QB, NB), 1)
    v = None
    for t in range(TOPK):
        v = jnp.max(m, axis=1, keepdims=True)
        sel = m == v
        bid = jnp.min(jnp.where(sel, iota, jnp.int32(1 << 30)),
                      axis=1, keepdims=True)
        bids_ref[:, t:t + 1] = bid
        m = jnp.where(iota == bid, NEG, m)
    tau_ref[...] = jnp.broadcast_to(v, (QB, 16))


def _buckets_call(m2):
    return pl.pallas_call(
        _buckets_body,
        grid=(Q // QB,),
        in_specs=[pl.BlockSpec((QB, NB), lambda i: (i, 0))],
        out_specs=[
            pl.BlockSpec((QB, TOPK), lambda i: (i, 0)),
            pl.BlockSpec((QB, 16), lambda i: (i, 0)),
        ],
        out_shape=[
            jax.ShapeDtypeStruct((Q, TOPK), jnp.int32),
            jax.ShapeDtypeStruct((Q, 16), jnp.float32),
        ],
    )(m2)


NW = 32
QPW = Q // NW
QBATCH = 8
NBATCH = QPW // QBATCH
CAP = TOPK * BUCKET + 32


def _sc_retrieve(scores2, bids, tau_rep, values):
    mesh = plsc.VectorSubcoreMesh(core_axis_name="c", subcore_axis_name="s")

    @functools.partial(
        pl.kernel, mesh=mesh,
        compiler_params=pltpu.CompilerParams(needs_layout_passes=False),
        out_type=jax.ShapeDtypeStruct((Q, D), jnp.float32),
        scratch_types=[
            pltpu.VMEM((QBATCH, TOPK), jnp.int32),
            pltpu.VMEM((QBATCH, 16), jnp.float32),
            pltpu.VMEM((2, 128), jnp.int32),
            pltpu.VMEM((QBATCH * TOPK, BUCKET), jnp.float32),
            pltpu.VMEM((CAP,), jnp.float32),
            pltpu.VMEM((CAP,), jnp.int32),
            pltpu.VMEM((QBATCH, TOPK), jnp.float32),
            pltpu.VMEM((2, 128), jnp.int32),
            pltpu.VMEM((QBATCH * TOPK, D), jnp.float32),
            pltpu.VMEM((QBATCH, D), jnp.float32),
            pltpu.SemaphoreType.DMA,
            pltpu.SemaphoreType.DMA,
        ],
    )
    def body(scores_hbm, bids_hbm, tau_hbm, values_hbm, out_hbm,
             bids_v, tau_v, gidx_v, sbuf, cand_v, cand_i,
             w_all, vidx_v, vals_v, out_v, sem0, sem1):
        wid = lax.axis_index("s") * 2 + lax.axis_index("c")
        q0w = wid * QPW
        lanes = lax.iota(jnp.int32, 16)

        def batch_body(b, _):
            q0 = q0w + b * QBATCH
            pltpu.sync_copy(bids_hbm.at[pl.ds(q0, QBATCH)], bids_v)
            pltpu.sync_copy(tau_hbm.at[pl.ds(q0, QBATCH)], tau_v)

            def gidx_body(i, _):
                for h in range(2):
                    bidc = bids_v[i, pl.ds(h * 16, 16)]
                    g = bidc * Q + (q0 + i)
                    p = i * TOPK + h * 16
                    gidx_v[p >> 7, pl.ds(p & 127, 16)] = g
                return 0
            lax.fori_loop(0, QBATCH, gidx_body, 0)

            cp0 = pltpu.async_copy(
                scores_hbm.at[gidx_v.at[0]], sbuf.at[pl.ds(0, 128)], sem0)
            cp1 = pltpu.async_copy(
                scores_hbm.at[gidx_v.at[1]], sbuf.at[pl.ds(128, 128)], sem0)
            cp0.wait()
            cp1.wait()

            def query_body(i, _):
                tau_s = tau_v[i, pl.ds(0, 16)][0]

                def row_body(row, cnt):
                    base = row * BUCKET
                    vs, ms, ps = [], [], []
                    for sub in range(8):
                        v = sbuf[i * TOPK + row, pl.ds(sub * 16, 16)]
                        m = v >= tau_s
                        vs.append(v)
                        ms.append(m)
                        ps.append(plsc.all_reduce_population_count(m)[0])
                    offs = []
                    o = cnt
                    for sub in range(8):
                        offs.append(o)
                        o = o + ps[sub]
                    for sub in range(8):
                        plsc.store_compressed(
                            cand_v.at[pl.ds(offs[sub], 16)], vs[sub],
                            mask=ms[sub])
                        plsc.store_compressed(
                            cand_i.at[pl.ds(offs[sub], 16)],
                            base + sub * 16 + lanes, mask=ms[sub])
                    return o
                cnt = lax.fori_loop(0, TOPK, row_body, jnp.int32(0))
                negv = jnp.full((16,), NEG)
                cand_v[pl.ds(cnt, 16)] = negv
                cand_v[pl.ds(cnt + 16, 16)] = negv

                nch = (cnt + 15) >> 4

                def vrev(x):
                    return lax.rev(x, (0,))

                def sel_body(ch, carry):
                    r0k, r0v, r1k, r1v = carry
                    sk = cand_v[pl.ds(ch * 16, 16)]
                    sv = cand_i[pl.ds(ch * 16, 16)]
                    sk, sv = plsc.sort_key_val(sk, sv, descending=True)
                    rsk, rsv = vrev(sk), vrev(sv)
                    take = r1k >= rsk
                    t1k = jnp.where(take, r1k, rsk)
                    t1v = jnp.where(take, r1v, rsv)
                    take2 = r0k >= t1k
                    u0k = jnp.where(take2, r0k, t1k)
                    u0v = jnp.where(take2, r0v, t1v)
                    u1k = jnp.where(take2, t1k, r0k)
                    u1v = jnp.where(take2, t1v, r0v)
                    u0k, u0v = plsc.sort_key_val(u0k, u0v, descending=True)
                    u1k, u1v = plsc.sort_key_val(u1k, u1v, descending=True)
                    return u0k, u0v, u1k, u1v
                tvA, tiA, tvB, tiB = lax.fori_loop(
                    0, nch, sel_body,
                    (negv, jnp.zeros((16,), jnp.int32),
                     negv, jnp.zeros((16,), jnp.int32)))

                mx = jnp.max(tvA)
                e0 = jnp.exp(tvA - mx)
                e1 = jnp.exp(tvB - mx)
                tot = jnp.sum(e0 + e1)
                w_all[i, pl.ds(0, 16)] = e0 / tot
                w_all[i, pl.ds(16, 16)] = e1 / tot
                iv = jnp.full((16,), i)
                g0 = plsc.load_gather(
                    bids_v, [iv, tiA >> 7]) * BUCKET + (tiA & 127)
                g1 = plsc.load_gather(
                    bids_v, [iv, tiB >> 7]) * BUCKET + (tiB & 127)
                p = i * TOPK
                vidx_v[p >> 7, pl.ds(p & 127, 16)] = g0
                p = i * TOPK + 16
                vidx_v[p >> 7, pl.ds(p & 127, 16)] = g1
                return 0
            lax.fori_loop(0, QBATCH, query_body, 0)

            cpv0 = pltpu.async_copy(
                values_hbm.at[vidx_v.at[0]], vals_v.at[pl.ds(0, 128)], sem1)
            cpv1 = pltpu.async_copy(
                values_hbm.at[vidx_v.at[1]], vals_v.at[pl.ds(128, 128)], sem1)
            cpv0.wait()
            cpv1.wait()

            def wsum_body(i, _):
                def acc_body(j, acc):
                    wvec = w_all[i, pl.ds((j >> 4) * 16, 16)]
                    ws = jnp.sum(jnp.where(lanes == (j & 15), wvec, 0.0))
                    row = i * TOPK + j
                    return tuple(
                        acc[c] + ws * vals_v[row, pl.ds(c * 16, 16)]
                        for c in range(D // 16))
                acc = lax.fori_loop(
                    0, TOPK, acc_body,
                    tuple(jnp.zeros((16,), jnp.float32)
                          for _ in range(D // 16)),
                    unroll=4)
                for c in range(D // 16):
                    out_v[i, pl.ds(c * 16, 16)] = acc[c]
                return 0
            lax.fori_loop(0, QBATCH, wsum_body, 0)

            pltpu.sync_copy(out_v, out_hbm.at[pl.ds(q0, QBATCH)])
            return 0

        lax.fori_loop(0, NBATCH, batch_body, 0)

    return body(scores2, bids, tau_rep, values)


def kernel(query, mem_keys, mem_values, importance, Wq, bq, top_k):
    query = query.astype(jnp.float32)
    mem_keys = mem_keys.astype(jnp.float32)
    bq2 = bq.reshape(1, D).astype(jnp.float32)

    q, qn = pl.pallas_call(
        _proj_body,
        in_specs=[
            pl.BlockSpec((Q, D), lambda: (0, 0)),
            pl.BlockSpec((D, D), lambda: (0, 0)),
            pl.BlockSpec((1, D), lambda: (0, 0)),
        ],
        out_specs=[
            pl.BlockSpec((Q, D), lambda: (0, 0)),
            pl.BlockSpec((Q, 1), lambda: (0, 0)),
        ],
        out_shape=[
            jax.ShapeDtypeStruct((Q, D), jnp.float32),
            jax.ShapeDtypeStruct((Q, 1), jnp.float32),
        ],
    )(query, Wq, bq2)

    imp_pad = jnp.pad(importance.astype(jnp.float32), (0, KPAD - K)).reshape(1, KPAD)

    scores2, m3 = _scores_call(q, qn, mem_keys, imp_pad)

    m2 = m3.transpose(1, 0, 2).reshape(Q, NB)
    bids, tau_rep = _buckets_call(m2)

    return _sc_retrieve(scores2, bids, tau_rep,
                        mem_values.astype(jnp.float32))


# speedup vs baseline: 21.9273x; 1.0261x over previous
import functools

import jax
import jax.numpy as jnp
from jax import lax
from jax.experimental import pallas as pl
from jax.experimental.pallas import tpu as pltpu
from jax.experimental.pallas import tpu_sc as plsc

Q = 1024
D = 256
K = 100000
TOPK = 32
BUCKET = 128
KT = 2048
NB = 784
KPAD = NB * BUCKET
NKT = KPAD // KT
BPT = KT // BUCKET


def _proj_body(query_ref, wq_ref, bq_ref, q_ref, qn_ref):
    qp = jax.lax.dot_general(
        query_ref[...], wq_ref[...], (((1,), (1,)), ((), ())),
        preferred_element_type=jnp.float32) + bq_ref[...]
    q_ref[...] = qp
    qn = jnp.sqrt(jnp.sum(qp * qp, axis=1, keepdims=True))
    qn_ref[...] = 1.0 / jnp.maximum(qn, 1e-8)


def _scores_body(q_ref, rqn_ref, keys_ref, imp_ref, s_ref, m_ref):
    k_idx = pl.program_id(0)
    keys = keys_ref[...]
    num = jax.lax.dot_general(
        q_ref[...], keys, (((1,), (1,)), ((), ())),
        preferred_element_type=jnp.float32)
    kn2 = jax.lax.dot_general(
        jnp.ones((1, D), jnp.float32), keys * keys, (((1,), (1,)), ((), ())),
        preferred_element_type=jnp.float32,
        precision=jax.lax.Precision.HIGHEST)
    kn = jnp.sqrt(kn2)
    c = imp_ref[...] / jnp.maximum(kn, 1e-8)
    s = num * rqn_ref[...] * c
    jglob = k_idx * KT + jax.lax.broadcasted_iota(jnp.int32, (1, KT), 1)
    s = jnp.where(jglob < K, s, 0.0)
    cols = []
    for b in range(BPT):
        blk = s[:, b * BUCKET:(b + 1) * BUCKET]
        s_ref[pl.ds(b * Q, Q), :] = blk
        cols.append(jnp.max(blk, axis=1, keepdims=True))
    m_ref[0] = jnp.concatenate(cols, axis=1)


def _scores_call(q, qn, keys_pad, imp_pad):
    return pl.pallas_call(
        _scores_body,
        grid=(NKT,),
        in_specs=[
            pl.BlockSpec((Q, D), lambda k: (0, 0)),
            pl.BlockSpec((Q, 1), lambda k: (0, 0)),
            pl.BlockSpec((KT, D), lambda k: (k, 0)),
            pl.BlockSpec((1, KT), lambda k: (0, k)),
        ],
        out_specs=[
            pl.BlockSpec((BPT * Q, BUCKET), lambda k: (k, 0)),
            pl.BlockSpec((1, Q, BPT), lambda k: (k, 0, 0)),
        ],
        out_shape=[
            jax.ShapeDtypeStruct((NB * Q, BUCKET), jnp.float32),
            jax.ShapeDtypeStruct((NKT, Q, BPT), jnp.float32),
        ],
    )(q, qn, keys_pad, imp_pad)


QB = 256
NEG = -3e38


def _buckets_body(m_ref, bids_ref, tau_ref):
    m = m_ref[...]
    iota = lax.broadcasted_iota(jnp.int32, (QB, NB), 1)
    v = None
    for t in range(TOPK):
        v = jnp.max(m, axis=1, keepdims=True)
        sel = m == v
        bid = jnp.min(jnp.where(sel, iota, jnp.int32(1 << 30)),
                      axis=1, keepdims=True)
        bids_ref[:, t:t + 1] = bid
        m = jnp.where(iota == bid, NEG, m)
    tau_ref[...] = jnp.broadcast_to(v, (QB, 16))


def _buckets_call(m2):
    return pl.pallas_call(
        _buckets_body,
        grid=(Q // QB,),
        in_specs=[pl.BlockSpec((QB, NB), lambda i: (i, 0))],
        out_specs=[
            pl.BlockSpec((QB, TOPK), lambda i: (i, 0)),
            pl.BlockSpec((QB, 16), lambda i: (i, 0)),
        ],
        out_shape=[
            jax.ShapeDtypeStruct((Q, TOPK), jnp.int32),
            jax.ShapeDtypeStruct((Q, 16), jnp.float32),
        ],
    )(m2)


NW = 32
QPW = Q // NW
QBATCH = 8
NBATCH = QPW // QBATCH
CAP = TOPK * BUCKET + 32


def _sc_retrieve(scores2, bids, tau_rep, values):
    mesh = plsc.VectorSubcoreMesh(core_axis_name="c", subcore_axis_name="s")

    @functools.partial(
        pl.kernel, mesh=mesh,
        compiler_params=pltpu.CompilerParams(needs_layout_passes=False),
        out_type=jax.ShapeDtypeStruct((Q, D), jnp.float32),
        scratch_types=[
            pltpu.VMEM((QBATCH, TOPK), jnp.int32),
            pltpu.VMEM((QBATCH, 16), jnp.float32),
            pltpu.VMEM((2, 128), jnp.int32),
            pltpu.VMEM((QBATCH * TOPK, BUCKET), jnp.float32),
            pltpu.VMEM((CAP,), jnp.float32),
            pltpu.VMEM((CAP,), jnp.int32),
            pltpu.VMEM((QBATCH, TOPK), jnp.float32),
            pltpu.VMEM((2, 128), jnp.int32),
            pltpu.VMEM((QBATCH * TOPK, D), jnp.float32),
            pltpu.VMEM((QBATCH, D), jnp.float32),
            pltpu.SemaphoreType.DMA,
            pltpu.SemaphoreType.DMA,
        ],
    )
    def body(scores_hbm, bids_hbm, tau_hbm, values_hbm, out_hbm,
             bids_v, tau_v, gidx_v, sbuf, cand_v, cand_i,
             w_all, vidx_v, vals_v, out_v, sem0, sem1):
        wid = lax.axis_index("s") * 2 + lax.axis_index("c")
        q0w = wid * QPW
        lanes = lax.iota(jnp.int32, 16)

        def prep(b):
            q0 = q0w + b * QBATCH
            pltpu.sync_copy(bids_hbm.at[pl.ds(q0, QBATCH)], bids_v)
            pltpu.sync_copy(tau_hbm.at[pl.ds(q0, QBATCH)], tau_v)

            def gidx_body(i, _):
                for h in range(2):
                    bidc = bids_v[i, pl.ds(h * 16, 16)]
                    g = bidc * Q + (q0 + i)
                    p = i * TOPK + h * 16
                    gidx_v[p >> 7, pl.ds(p & 127, 16)] = g
                return 0
            lax.fori_loop(0, QBATCH, gidx_body, 0)

            pltpu.async_copy(
                scores_hbm.at[gidx_v.at[0]], sbuf.at[pl.ds(0, 128)], sem0)
            pltpu.async_copy(
                scores_hbm.at[gidx_v.at[1]], sbuf.at[pl.ds(128, 128)], sem0)

        prep(jnp.int32(0))

        def batch_body(b, _):
            q0 = q0w + b * QBATCH
            pltpu.make_async_copy(
                scores_hbm.at[gidx_v.at[0]], sbuf.at[pl.ds(0, 128)],
                sem0).wait()
            pltpu.make_async_copy(
                scores_hbm.at[gidx_v.at[1]], sbuf.at[pl.ds(128, 128)],
                sem0).wait()

            def query_body(i, _):
                tau_s = tau_v[i, pl.ds(0, 16)][0]

                def row_body(row, cnt):
                    base = row * BUCKET
                    vs, ms, ps = [], [], []
                    for sub in range(8):
                        v = sbuf[i * TOPK + row, pl.ds(sub * 16, 16)]
                        m = v >= tau_s
                        vs.append(v)
                        ms.append(m)
                        ps.append(plsc.all_reduce_population_count(m)[0])
                    offs = []
                    o = cnt
                    for sub in range(8):
                        offs.append(o)
                        o = o + ps[sub]
                    for sub in range(8):
                        plsc.store_compressed(
                            cand_v.at[pl.ds(offs[sub], 16)], vs[sub],
                            mask=ms[sub])
                        plsc.store_compressed(
                            cand_i.at[pl.ds(offs[sub], 16)],
                            base + sub * 16 + lanes, mask=ms[sub])
                    return o
                cnt = lax.fori_loop(0, TOPK, row_body, jnp.int32(0))
                negv = jnp.full((16,), NEG)
                cand_v[pl.ds(cnt, 16)] = negv
                cand_v[pl.ds(cnt + 16, 16)] = negv

                nch = (cnt + 15) >> 4

                def vrev(x):
                    return lax.rev(x, (0,))

                def sel_body(ch, carry):
                    r0k, r0v, r1k, r1v = carry
                    sk = cand_v[pl.ds(ch * 16, 16)]
                    sv = cand_i[pl.ds(ch * 16, 16)]
                    sk, sv = plsc.sort_key_val(sk, sv, descending=True)
                    rsk, rsv = vrev(sk), vrev(sv)
                    take = r1k >= rsk
                    t1k = jnp.where(take, r1k, rsk)
                    t1v = jnp.where(take, r1v, rsv)
                    take2 = r0k >= t1k
                    u0k = jnp.where(take2, r0k, t1k)
                    u0v = jnp.where(take2, r0v, t1v)
                    u1k = jnp.where(take2, t1k, r0k)
                    u1v = jnp.where(take2, t1v, r0v)
                    u0k, u0v = plsc.sort_key_val(u0k, u0v, descending=True)
                    u1k, u1v = plsc.sort_key_val(u1k, u1v, descending=True)
                    return u0k, u0v, u1k, u1v
                tvA, tiA, tvB, tiB = lax.fori_loop(
                    0, nch, sel_body,
                    (negv, jnp.zeros((16,), jnp.int32),
                     negv, jnp.zeros((16,), jnp.int32)))

                mx = jnp.max(tvA)
                e0 = jnp.exp(tvA - mx)
                e1 = jnp.exp(tvB - mx)
                tot = jnp.sum(e0 + e1)
                w_all[i, pl.ds(0, 16)] = e0 / tot
                w_all[i, pl.ds(16, 16)] = e1 / tot
                iv = jnp.full((16,), i)
                g0 = plsc.load_gather(
                    bids_v, [iv, tiA >> 7]) * BUCKET + (tiA & 127)
                g1 = plsc.load_gather(
                    bids_v, [iv, tiB >> 7]) * BUCKET + (tiB & 127)
                p = i * TOPK
                vidx_v[p >> 7, pl.ds(p & 127, 16)] = g0
                p = i * TOPK + 16
                vidx_v[p >> 7, pl.ds(p & 127, 16)] = g1
                return 0
            lax.fori_loop(0, QBATCH, query_body, 0)

            pltpu.async_copy(
                values_hbm.at[vidx_v.at[0]], vals_v.at[pl.ds(0, 128)], sem1)
            pltpu.async_copy(
                values_hbm.at[vidx_v.at[1]], vals_v.at[pl.ds(128, 128)], sem1)

            @pl.when(b < NBATCH - 1)
            def _():
                prep(b + 1)

            pltpu.make_async_copy(
                values_hbm.at[vidx_v.at[0]], vals_v.at[pl.ds(0, 128)],
                sem1).wait()
            pltpu.make_async_copy(
                values_hbm.at[vidx_v.at[1]], vals_v.at[pl.ds(128, 128)],
                sem1).wait()

            def wsum_body(i, _):
                def acc_body(j, acc):
                    wvec = w_all[i, pl.ds((j >> 4) * 16, 16)]
                    ws = jnp.sum(jnp.where(lanes == (j & 15), wvec, 0.0))
                    row = i * TOPK + j
                    return tuple(
                        acc[c] + ws * vals_v[row, pl.ds(c * 16, 16)]
                        for c in range(D // 16))
                acc = lax.fori_loop(
                    0, TOPK, acc_body,
                    tuple(jnp.zeros((16,), jnp.float32)
                          for _ in range(D // 16)),
                    unroll=4)
                for c in range(D // 16):
                    out_v[i, pl.ds(c * 16, 16)] = acc[c]
                return 0
            lax.fori_loop(0, QBATCH, wsum_body, 0)

            pltpu.sync_copy(out_v, out_hbm.at[pl.ds(q0, QBATCH)])
            return 0

        lax.fori_loop(0, NBATCH, batch_body, 0)

    return body(scores2, bids, tau_rep, values)


def kernel(query, mem_keys, mem_values, importance, Wq, bq, top_k):
    query = query.astype(jnp.float32)
    mem_keys = mem_keys.astype(jnp.float32)
    bq2 = bq.reshape(1, D).astype(jnp.float32)

    q, qn = pl.pallas_call(
        _proj_body,
        in_specs=[
            pl.BlockSpec((Q, D), lambda: (0, 0)),
            pl.BlockSpec((D, D), lambda: (0, 0)),
            pl.BlockSpec((1, D), lambda: (0, 0)),
        ],
        out_specs=[
            pl.BlockSpec((Q, D), lambda: (0, 0)),
            pl.BlockSpec((Q, 1), lambda: (0, 0)),
        ],
        out_shape=[
            jax.ShapeDtypeStruct((Q, D), jnp.float32),
            jax.ShapeDtypeStruct((Q, 1), jnp.float32),
        ],
    )(query, Wq, bq2)

    imp_pad = jnp.pad(importance.astype(jnp.float32), (0, KPAD - K)).reshape(1, KPAD)

    scores2, m3 = _scores_call(q, qn, mem_keys, imp_pad)

    m2 = m3.transpose(1, 0, 2).reshape(Q, NB)
    bids, tau_rep = _buckets_call(m2)

    return _sc_retrieve(scores2, bids, tau_rep,
                        mem_values.astype(jnp.float32))


# speedup vs baseline: 21.9531x; 1.0012x over previous
import functools

import jax
import jax.numpy as jnp
from jax import lax
from jax.experimental import pallas as pl
from jax.experimental.pallas import tpu as pltpu
from jax.experimental.pallas import tpu_sc as plsc

Q = 1024
D = 256
K = 100000
TOPK = 32
BUCKET = 128
KT = 2048
NB = 784
KPAD = NB * BUCKET
NKT = KPAD // KT
BPT = KT // BUCKET


def _scores_body(query_ref, wq_ref, bq_ref, keys_ref, imp_ref, s_ref, m_ref):
    k_idx = pl.program_id(0)
    qp = jax.lax.dot_general(
        query_ref[...], wq_ref[...], (((1,), (1,)), ((), ())),
        preferred_element_type=jnp.float32) + bq_ref[...]
    qn = jnp.sqrt(jnp.sum(qp * qp, axis=1, keepdims=True))
    rqn = 1.0 / jnp.maximum(qn, 1e-8)
    keys = keys_ref[...]
    num = jax.lax.dot_general(
        qp, keys, (((1,), (1,)), ((), ())),
        preferred_element_type=jnp.float32)
    kn2 = jax.lax.dot_general(
        jnp.ones((1, D), jnp.float32), keys * keys, (((1,), (1,)), ((), ())),
        preferred_element_type=jnp.float32,
        precision=jax.lax.Precision.HIGHEST)
    kn = jnp.sqrt(kn2)
    c = imp_ref[...] / jnp.maximum(kn, 1e-8)
    s = num * rqn * c
    jglob = k_idx * KT + jax.lax.broadcasted_iota(jnp.int32, (1, KT), 1)
    s = jnp.where(jglob < K, s, 0.0)
    cols = []
    for b in range(BPT):
        blk = s[:, b * BUCKET:(b + 1) * BUCKET]
        s_ref[pl.ds(b * Q, Q), :] = blk
        cols.append(jnp.max(blk, axis=1, keepdims=True))
    m_ref[0] = jnp.concatenate(cols, axis=1)


def _scores_call(query, wq, bq2, keys, imp_pad):
    return pl.pallas_call(
        _scores_body,
        grid=(NKT,),
        in_specs=[
            pl.BlockSpec((Q, D), lambda k: (0, 0)),
            pl.BlockSpec((D, D), lambda k: (0, 0)),
            pl.BlockSpec((1, D), lambda k: (0, 0)),
            pl.BlockSpec((KT, D), lambda k: (k, 0)),
            pl.BlockSpec((1, KT), lambda k: (0, k)),
        ],
        out_specs=[
            pl.BlockSpec((BPT * Q, BUCKET), lambda k: (k, 0)),
            pl.BlockSpec((1, Q, BPT), lambda k: (k, 0, 0)),
        ],
        out_shape=[
            jax.ShapeDtypeStruct((NB * Q, BUCKET), jnp.float32),
            jax.ShapeDtypeStruct((NKT, Q, BPT), jnp.float32),
        ],
    )(query, wq, bq2, keys, imp_pad)


QB = 256
NEG = -3e38


def _buckets_body(m_ref, bids_ref, tau_ref):
    m = m_ref[...]
    iota = lax.broadcasted_iota(jnp.int32, (QB, NB), 1)
    v = None
    for t in range(TOPK):
        v = jnp.max(m, axis=1, keepdims=True)
        sel = m == v
        bid = jnp.min(jnp.where(sel, iota, jnp.int32(1 << 30)),
                      axis=1, keepdims=True)
        bids_ref[:, t:t + 1] = bid
        m = jnp.where(iota == bid, NEG, m)
    tau_ref[...] = jnp.broadcast_to(v, (QB, 16))


def _buckets_call(m2):
    return pl.pallas_call(
        _buckets_body,
        grid=(Q // QB,),
        in_specs=[pl.BlockSpec((QB, NB), lambda i: (i, 0))],
        out_specs=[
            pl.BlockSpec((QB, TOPK), lambda i: (i, 0)),
            pl.BlockSpec((QB, 16), lambda i: (i, 0)),
        ],
        out_shape=[
            jax.ShapeDtypeStruct((Q, TOPK), jnp.int32),
            jax.ShapeDtypeStruct((Q, 16), jnp.float32),
        ],
    )(m2)


NW = 32
QPW = Q // NW
QBATCH = 8
NBATCH = QPW // QBATCH
CAP = TOPK * BUCKET + 32


def _sc_retrieve(scores2, bids, tau_rep, values):
    mesh = plsc.VectorSubcoreMesh(core_axis_name="c", subcore_axis_name="s")

    @functools.partial(
        pl.kernel, mesh=mesh,
        compiler_params=pltpu.CompilerParams(needs_layout_passes=False),
        out_type=jax.ShapeDtypeStruct((Q, D), jnp.float32),
        scratch_types=[
            pltpu.VMEM((QBATCH, TOPK), jnp.int32),
            pltpu.VMEM((QBATCH, 16), jnp.float32),
            pltpu.VMEM((2, 128), jnp.int32),
            pltpu.VMEM((QBATCH * TOPK, BUCKET), jnp.float32),
            pltpu.VMEM((CAP,), jnp.float32),
            pltpu.VMEM((CAP,), jnp.int32),
            pltpu.VMEM((QBATCH, TOPK), jnp.float32),
            pltpu.VMEM((2, 128), jnp.int32),
            pltpu.VMEM((QBATCH * TOPK, D), jnp.float32),
            pltpu.VMEM((QBATCH, D), jnp.float32),
            pltpu.SemaphoreType.DMA,
            pltpu.SemaphoreType.DMA,
        ],
    )
    def body(scores_hbm, bids_hbm, tau_hbm, values_hbm, out_hbm,
             bids_v, tau_v, gidx_v, sbuf, cand_v, cand_i,
             w_all, vidx_v, vals_v, out_v, sem0, sem1):
        wid = lax.axis_index("s") * 2 + lax.axis_index("c")
        q0w = wid * QPW
        lanes = lax.iota(jnp.int32, 16)

        def prep(b):
            q0 = q0w + b * QBATCH
            pltpu.sync_copy(bids_hbm.at[pl.ds(q0, QBATCH)], bids_v)
            pltpu.sync_copy(tau_hbm.at[pl.ds(q0, QBATCH)], tau_v)

            def gidx_body(i, _):
                for h in range(2):
                    bidc = bids_v[i, pl.ds(h * 16, 16)]
                    g = bidc * Q + (q0 + i)
                    p = i * TOPK + h * 16
                    gidx_v[p >> 7, pl.ds(p & 127, 16)] = g
                return 0
            lax.fori_loop(0, QBATCH, gidx_body, 0)

            pltpu.async_copy(
                scores_hbm.at[gidx_v.at[0]], sbuf.at[pl.ds(0, 128)], sem0)
            pltpu.async_copy(
                scores_hbm.at[gidx_v.at[1]], sbuf.at[pl.ds(128, 128)], sem0)

        prep(jnp.int32(0))

        def batch_body(b, _):
            q0 = q0w + b * QBATCH
            pltpu.make_async_copy(
                scores_hbm.at[gidx_v.at[0]], sbuf.at[pl.ds(0, 128)],
                sem0).wait()
            pltpu.make_async_copy(
                scores_hbm.at[gidx_v.at[1]], sbuf.at[pl.ds(128, 128)],
                sem0).wait()

            def query_body(i, _):
                tau_s = tau_v[i, pl.ds(0, 16)][0]

                def row_body(row, cnt):
                    base = row * BUCKET
                    vs, ms, ps = [], [], []
                    for sub in range(8):
                        v = sbuf[i * TOPK + row, pl.ds(sub * 16, 16)]
                        m = v >= tau_s
                        vs.append(v)
                        ms.append(m)
                        ps.append(plsc.all_reduce_population_count(m)[0])
                    offs = []
                    o = cnt
                    for sub in range(8):
                        offs.append(o)
                        o = o + ps[sub]
                    for sub in range(8):
                        plsc.store_compressed(
                            cand_v.at[pl.ds(offs[sub], 16)], vs[sub],
                            mask=ms[sub])
                        plsc.store_compressed(
                            cand_i.at[pl.ds(offs[sub], 16)],
                            base + sub * 16 + lanes, mask=ms[sub])
                    return o
                cnt = lax.fori_loop(0, TOPK, row_body, jnp.int32(0))
                negv = jnp.full((16,), NEG)
                cand_v[pl.ds(cnt, 16)] = negv
                cand_v[pl.ds(cnt + 16, 16)] = negv

                nch = (cnt + 15) >> 4

                def vrev(x):
                    return lax.rev(x, (0,))

                def sel_body(ch, carry):
                    r0k, r0v, r1k, r1v = carry
                    sk = cand_v[pl.ds(ch * 16, 16)]
                    sv = cand_i[pl.ds(ch * 16, 16)]
                    sk, sv = plsc.sort_key_val(sk, sv, descending=True)
                    rsk, rsv = vrev(sk), vrev(sv)
                    take = r1k >= rsk
                    t1k = jnp.where(take, r1k, rsk)
                    t1v = jnp.where(take, r1v, rsv)
                    take2 = r0k >= t1k
                    u0k = jnp.where(take2, r0k, t1k)
                    u0v = jnp.where(take2, r0v, t1v)
                    u1k = jnp.where(take2, t1k, r0k)
                    u1v = jnp.where(take2, t1v, r0v)
                    u0k, u0v = plsc.sort_key_val(u0k, u0v, descending=True)
                    u1k, u1v = plsc.sort_key_val(u1k, u1v, descending=True)
                    return u0k, u0v, u1k, u1v
                tvA, tiA, tvB, tiB = lax.fori_loop(
                    0, nch, sel_body,
                    (negv, jnp.zeros((16,), jnp.int32),
                     negv, jnp.zeros((16,), jnp.int32)))

                mx = jnp.max(tvA)
                e0 = jnp.exp(tvA - mx)
                e1 = jnp.exp(tvB - mx)
                tot = jnp.sum(e0 + e1)
                w_all[i, pl.ds(0, 16)] = e0 / tot
                w_all[i, pl.ds(16, 16)] = e1 / tot
                iv = jnp.full((16,), i)
                g0 = plsc.load_gather(
                    bids_v, [iv, tiA >> 7]) * BUCKET + (tiA & 127)
                g1 = plsc.load_gather(
                    bids_v, [iv, tiB >> 7]) * BUCKET + (tiB & 127)
                p = i * TOPK
                vidx_v[p >> 7, pl.ds(p & 127, 16)] = g0
                p = i * TOPK + 16
                vidx_v[p >> 7, pl.ds(p & 127, 16)] = g1
                return 0
            lax.fori_loop(0, QBATCH, query_body, 0)

            pltpu.async_copy(
                values_hbm.at[vidx_v.at[0]], vals_v.at[pl.ds(0, 128)], sem1)
            pltpu.async_copy(
                values_hbm.at[vidx_v.at[1]], vals_v.at[pl.ds(128, 128)], sem1)

            @pl.when(b < NBATCH - 1)
            def _():
                prep(b + 1)

            pltpu.make_async_copy(
                values_hbm.at[vidx_v.at[0]], vals_v.at[pl.ds(0, 128)],
                sem1).wait()
            pltpu.make_async_copy(
                values_hbm.at[vidx_v.at[1]], vals_v.at[pl.ds(128, 128)],
                sem1).wait()

            def wsum_body(i, _):
                def acc_body(j, acc):
                    wvec = w_all[i, pl.ds((j >> 4) * 16, 16)]
                    ws = jnp.sum(jnp.where(lanes == (j & 15), wvec, 0.0))
                    row = i * TOPK + j
                    return tuple(
                        acc[c] + ws * vals_v[row, pl.ds(c * 16, 16)]
                        for c in range(D // 16))
                acc = lax.fori_loop(
                    0, TOPK, acc_body,
                    tuple(jnp.zeros((16,), jnp.float32)
                          for _ in range(D // 16)),
                    unroll=4)
                for c in range(D // 16):
                    out_v[i, pl.ds(c * 16, 16)] = acc[c]
                return 0
            lax.fori_loop(0, QBATCH, wsum_body, 0)

            pltpu.sync_copy(out_v, out_hbm.at[pl.ds(q0, QBATCH)])
            return 0

        lax.fori_loop(0, NBATCH, batch_body, 0)

    return body(scores2, bids, tau_rep, values)


def kernel(query, mem_keys, mem_values, importance, Wq, bq, top_k):
    query = query.astype(jnp.float32)
    mem_keys = mem_keys.astype(jnp.float32)
    bq2 = bq.reshape(1, D).astype(jnp.float32)

    imp_pad = jnp.pad(importance.astype(jnp.float32), (0, KPAD - K)).reshape(1, KPAD)

    scores2, m3 = _scores_call(query, Wq.astype(jnp.float32), bq2,
                               mem_keys, imp_pad)

    m2 = m3.transpose(1, 0, 2).reshape(Q, NB)
    bids, tau_rep = _buckets_call(m2)

    return _sc_retrieve(scores2, bids, tau_rep,
                        mem_values.astype(jnp.float32))
